# Initial kernel scaffold; baseline (speedup 1.0000x reference)
import jax
import jax.numpy as jnp
from jax.experimental import pallas as pl


def kernel(x, edge_index, W1, b1, W2, b2, W3, b3):
    raise NotImplementedError("write your pallas kernel here")


# speedup vs baseline: 4.4499x; 4.4499x over previous
import functools

import jax
import jax.numpy as jnp
from jax import lax
from jax.experimental import pallas as pl
from jax.experimental.pallas import tpu as pltpu
from jax.experimental.pallas import tpu_sc as plsc

N = 10000
E = 320000
NC = 2
NS = 16
NW = NC * NS
CHE = 128
EPT = 10240
NCH = EPT // CHE
NPH = 2
E_PAD = NW * EPT
N_PAD = 10112
ZPT = N_PAD // NS


def _make_agg(D):
  mesh = plsc.VectorSubcoreMesh(core_axis_name="c", subcore_axis_name="s")

  @functools.partial(
      pl.kernel,
      out_type=jax.ShapeDtypeStruct((NC, N_PAD, D), jnp.float32),
      mesh=mesh,
      compiler_params=pltpu.CompilerParams(use_tc_tiling_on_sc=False),
      scratch_types=[
          pltpu.VMEM((NCH // NPH, CHE), jnp.int32),
          pltpu.VMEM((NCH // NPH, CHE), jnp.int32),
          pltpu.VMEM((2, CHE, D), jnp.float32),
          pltpu.VMEM_SHARED((N_PAD, D), jnp.float32),
          pltpu.SemaphoreType.DMA,
          pltpu.SemaphoreType.DMA,
      ],
  )
  def agg(g_hbm, src_hbm, dst_hbm, zeros_hbm, out_hbm,
          srcv, dstv, rows, acc, sem0, sem1):
    c = lax.axis_index("c")
    s = lax.axis_index("s")
    w = c * NS + s
    nch_p = NCH // NPH

    pltpu.sync_copy(zeros_hbm, acc.at[pl.ds(s * ZPT, ZPT)])
    plsc.subcore_barrier()

    for p in range(NPH):
      pltpu.sync_copy(src_hbm.at[w, pl.ds(p * nch_p, nch_p)], srcv)
      pltpu.sync_copy(dst_hbm.at[w, pl.ds(p * nch_p, nch_p)], dstv)
      pltpu.async_copy(g_hbm.at[srcv.at[0]], rows.at[0], sem0)

      def step(i, carry):
        k0 = i * 2
        k1 = k0 + 1
        pltpu.async_copy(g_hbm.at[srcv.at[k1]], rows.at[1], sem1)
        pltpu.make_async_copy(g_hbm.at[srcv.at[k0]], rows.at[0], sem0).wait()
        pltpu.sync_copy(rows.at[0], acc.at[dstv.at[k0]], add=True)
        pltpu.async_copy(g_hbm.at[srcv.at[k0 + 2]], rows.at[0], sem0)
        pltpu.make_async_copy(g_hbm.at[srcv.at[k1]], rows.at[1], sem1).wait()
        pltpu.sync_copy(rows.at[1], acc.at[dstv.at[k1]], add=True)
        return carry

      lax.fori_loop(0, nch_p // 2 - 1, step, 0)

      kl = nch_p - 2
      pltpu.async_copy(g_hbm.at[srcv.at[kl + 1]], rows.at[1], sem1)
      pltpu.make_async_copy(g_hbm.at[srcv.at[kl]], rows.at[0], sem0).wait()
      pltpu.sync_copy(rows.at[0], acc.at[dstv.at[kl]], add=True)
      pltpu.make_async_copy(g_hbm.at[srcv.at[kl + 1]], rows.at[1], sem1).wait()
      pltpu.sync_copy(rows.at[1], acc.at[dstv.at[kl + 1]], add=True)

    plsc.subcore_barrier()
    pltpu.sync_copy(acc.at[pl.ds(s * ZPT, ZPT)],
                    out_hbm.at[c, pl.ds(s * ZPT, ZPT)])

  return agg


_agg128 = _make_agg(128)
_agg64 = _make_agg(64)


def _mm_body(x_ref, w_ref, o_ref):
  o_ref[...] = jnp.dot(x_ref[...], w_ref[...],
                       preferred_element_type=jnp.float32)


def _combine_mm_body(g_ref, p0_ref, p1_ref, b_ref, w_ref, o_ref):
  h = g_ref[...] + p0_ref[0] + p1_ref[0] + b_ref[...]
  h = jnp.maximum(h, 0.0)
  o_ref[...] = jnp.dot(h, w_ref[...], preferred_element_type=jnp.float32)


def _combine_body(g_ref, p0_ref, p1_ref, b_ref, o_ref):
  o_ref[...] = g_ref[...] + p0_ref[0] + p1_ref[0] + b_ref[...]


_BM = 1000


def _mm(x, w):
  n, d = x.shape
  h = w.shape[1]
  return pl.pallas_call(
      _mm_body,
      grid=(n // _BM,),
      in_specs=[
          pl.BlockSpec((_BM, d), lambda i: (i, 0)),
          pl.BlockSpec((d, h), lambda i: (0, 0)),
      ],
      out_specs=pl.BlockSpec((_BM, h), lambda i: (i, 0)),
      out_shape=jax.ShapeDtypeStruct((n, h), jnp.float32),
  )(x, w)


def _combine_mm(g, p, b, w):
  n, d = g.shape
  h = w.shape[1]
  return pl.pallas_call(
      _combine_mm_body,
      grid=(n // _BM,),
      in_specs=[
          pl.BlockSpec((_BM, d), lambda i: (i, 0)),
          pl.BlockSpec((1, _BM, d), lambda i: (0, i, 0)),
          pl.BlockSpec((1, _BM, d), lambda i: (1, i, 0)),
          pl.BlockSpec((1, d), lambda i: (0, 0)),
          pl.BlockSpec((d, h), lambda i: (0, 0)),
      ],
      out_specs=pl.BlockSpec((_BM, h), lambda i: (i, 0)),
      out_shape=jax.ShapeDtypeStruct((n, h), jnp.float32),
  )(g, p, p, b.reshape(1, d), w)


def _combine(g, p, b):
  n, d = g.shape
  return pl.pallas_call(
      _combine_body,
      grid=(n // _BM,),
      in_specs=[
          pl.BlockSpec((_BM, d), lambda i: (i, 0)),
          pl.BlockSpec((1, _BM, d), lambda i: (0, i, 0)),
          pl.BlockSpec((1, _BM, d), lambda i: (1, i, 0)),
          pl.BlockSpec((1, d), lambda i: (0, 0)),
      ],
      out_specs=pl.BlockSpec((_BM, d), lambda i: (i, 0)),
      out_shape=jax.ShapeDtypeStruct((n, d), jnp.float32),
  )(g, p, p, b.reshape(1, d))


def kernel(x, edge_index, W1, b1, W2, b2, W3, b3):
  src = edge_index[0]
  dst = edge_index[1]
  npad = E_PAD - E
  srcp = jnp.concatenate([src, jnp.zeros((npad,), jnp.int32)]).reshape(
      NW, NCH, CHE)
  dstp = jnp.concatenate([dst, jnp.full((npad,), N, jnp.int32)]).reshape(
      NW, NCH, CHE)
  z128 = jnp.zeros((ZPT, 128), jnp.float32)
  z64 = jnp.zeros((ZPT, 64), jnp.float32)

  g1 = _mm(x, W1)
  p1 = _agg128(g1, srcp, dstp, z128)
  g2 = _combine_mm(g1, p1, b1, W2)
  p2 = _agg128(g2, srcp, dstp, z128)
  g3 = _combine_mm(g2, p2, b2, W3)
  p3 = _agg64(g3, srcp, dstp, z64)
  return _combine(g3, p3, b3)


# speedup vs baseline: 9.2650x; 2.0821x over previous
import functools

import jax
import jax.numpy as jnp
from jax import lax
from jax.experimental import pallas as pl
from jax.experimental.pallas import tpu as pltpu
from jax.experimental.pallas import tpu_sc as plsc

N = 10000
E = 320000
NC = 2
NS = 16
CHE = 128
EPT = 20480
NCHT = EPT // CHE
NPH = 4
E_PAD = NS * EPT
N_PAD = 10112
ZPT = N_PAD // NS
TPT = N // NS


def _make_agg(DH):
  mesh = plsc.VectorSubcoreMesh(core_axis_name="c", subcore_axis_name="s")

  @functools.partial(
      pl.kernel,
      out_type=jax.ShapeDtypeStruct((NC, N_PAD, DH), jnp.float32),
      mesh=mesh,
      compiler_params=pltpu.CompilerParams(use_tc_tiling_on_sc=False),
      scratch_types=[
          pltpu.VMEM((NCHT // NPH, CHE), jnp.int32),
          pltpu.VMEM((NCHT // NPH, CHE), jnp.int32),
          pltpu.VMEM((2, CHE, DH), jnp.float32),
          pltpu.VMEM_SHARED((N, DH), jnp.float32),
          pltpu.VMEM_SHARED((N_PAD, DH), jnp.float32),
          pltpu.SemaphoreType.DMA,
          pltpu.SemaphoreType.DMA,
      ],
  )
  def agg(g2_hbm, src_hbm, dst_hbm, zeros_hbm, out_hbm,
          srcv, dstv, rows, tab, acc, sem0, sem1):
    c = lax.axis_index("c")
    s = lax.axis_index("s")
    nch_p = NCHT // NPH

    pltpu.sync_copy(g2_hbm.at[c, pl.ds(s * TPT, TPT)],
                    tab.at[pl.ds(s * TPT, TPT)])
    pltpu.sync_copy(zeros_hbm, acc.at[pl.ds(s * ZPT, ZPT)])
    plsc.subcore_barrier()

    for p in range(NPH):
      pltpu.sync_copy(src_hbm.at[s, pl.ds(p * nch_p, nch_p)], srcv)
      pltpu.sync_copy(dst_hbm.at[s, pl.ds(p * nch_p, nch_p)], dstv)
      pltpu.async_copy(tab.at[srcv.at[0]], rows.at[0], sem0)

      def step(i, carry):
        k0 = i * 2
        k1 = k0 + 1
        pltpu.async_copy(tab.at[srcv.at[k1]], rows.at[1], sem1)
        pltpu.make_async_copy(tab.at[srcv.at[k0]], rows.at[0], sem0).wait()
        pltpu.sync_copy(rows.at[0], acc.at[dstv.at[k0]], add=True)
        pltpu.async_copy(tab.at[srcv.at[k0 + 2]], rows.at[0], sem0)
        pltpu.make_async_copy(tab.at[srcv.at[k1]], rows.at[1], sem1).wait()
        pltpu.sync_copy(rows.at[1], acc.at[dstv.at[k1]], add=True)
        return carry

      lax.fori_loop(0, nch_p // 2 - 1, step, 0)

      kl = nch_p - 2
      pltpu.async_copy(tab.at[srcv.at[kl + 1]], rows.at[1], sem1)
      pltpu.make_async_copy(tab.at[srcv.at[kl]], rows.at[0], sem0).wait()
      pltpu.sync_copy(rows.at[0], acc.at[dstv.at[kl]], add=True)
      pltpu.make_async_copy(tab.at[srcv.at[kl + 1]], rows.at[1], sem1).wait()
      pltpu.sync_copy(rows.at[1], acc.at[dstv.at[kl + 1]], add=True)

    plsc.subcore_barrier()
    pltpu.sync_copy(acc.at[pl.ds(s * ZPT, ZPT)],
                    out_hbm.at[c, pl.ds(s * ZPT, ZPT)])

  return agg


_agg64 = _make_agg(64)
_agg32 = _make_agg(32)


def _mm_split_body(x_ref, w_ref, o_ref):
  r = jnp.dot(x_ref[...], w_ref[...], preferred_element_type=jnp.float32)
  dh = r.shape[1] // 2
  o_ref[0] = r[:, :dh]
  o_ref[1] = r[:, dh:]


def _combine_mm_split_body(g_ref, p_ref, b_ref, w_ref, o_ref):
  h = jnp.concatenate([g_ref[0] + p_ref[0], g_ref[1] + p_ref[1]], axis=1)
  h = jnp.maximum(h + b_ref[...], 0.0)
  r = jnp.dot(h, w_ref[...], preferred_element_type=jnp.float32)
  dh = r.shape[1] // 2
  o_ref[0] = r[:, :dh]
  o_ref[1] = r[:, dh:]


def _combine_body(g_ref, p_ref, b_ref, o_ref):
  o_ref[...] = jnp.concatenate(
      [g_ref[0] + p_ref[0], g_ref[1] + p_ref[1]], axis=1) + b_ref[...]


_BM = 1000


def _mm_split(x, w):
  n, d = x.shape
  h = w.shape[1]
  return pl.pallas_call(
      _mm_split_body,
      grid=(n // _BM,),
      in_specs=[
          pl.BlockSpec((_BM, d), lambda i: (i, 0)),
          pl.BlockSpec((d, h), lambda i: (0, 0)),
      ],
      out_specs=pl.BlockSpec((2, _BM, h // 2), lambda i: (0, i, 0)),
      out_shape=jax.ShapeDtypeStruct((2, n, h // 2), jnp.float32),
  )(x, w)


def _combine_mm_split(g2, p, b, w):
  _, n, dh = g2.shape
  d = 2 * dh
  h = w.shape[1]
  return pl.pallas_call(
      _combine_mm_split_body,
      grid=(n // _BM,),
      in_specs=[
          pl.BlockSpec((2, _BM, dh), lambda i: (0, i, 0)),
          pl.BlockSpec((2, _BM, dh), lambda i: (0, i, 0)),
          pl.BlockSpec((1, d), lambda i: (0, 0)),
          pl.BlockSpec((d, h), lambda i: (0, 0)),
      ],
      out_specs=pl.BlockSpec((2, _BM, h // 2), lambda i: (0, i, 0)),
      out_shape=jax.ShapeDtypeStruct((2, n, h // 2), jnp.float32),
  )(g2, p, b.reshape(1, d), w)


def _combine(g2, p, b):
  _, n, dh = g2.shape
  d = 2 * dh
  return pl.pallas_call(
      _combine_body,
      grid=(n // _BM,),
      in_specs=[
          pl.BlockSpec((2, _BM, dh), lambda i: (0, i, 0)),
          pl.BlockSpec((2, _BM, dh), lambda i: (0, i, 0)),
          pl.BlockSpec((1, d), lambda i: (0, 0)),
      ],
      out_specs=pl.BlockSpec((_BM, d), lambda i: (i, 0)),
      out_shape=jax.ShapeDtypeStruct((n, d), jnp.float32),
  )(g2, p, b.reshape(1, d))


def kernel(x, edge_index, W1, b1, W2, b2, W3, b3):
  src = edge_index[0]
  dst = edge_index[1]
  npad = E_PAD - E
  srcp = jnp.concatenate([src, jnp.zeros((npad,), jnp.int32)]).reshape(
      NS, NCHT, CHE)
  dstp = jnp.concatenate([dst, jnp.full((npad,), N, jnp.int32)]).reshape(
      NS, NCHT, CHE)
  z64 = jnp.zeros((ZPT, 64), jnp.float32)
  z32 = jnp.zeros((ZPT, 32), jnp.float32)

  g1 = _mm_split(x, W1)
  p1 = _agg64(g1, srcp, dstp, z64)
  g2 = _combine_mm_split(g1, p1, b1, W2)
  p2 = _agg64(g2, srcp, dstp, z64)
  g3 = _combine_mm_split(g2, p2, b2, W3)
  p3 = _agg32(g3, srcp, dstp, z32)
  return _combine(g3, p3, b3)


# speedup vs baseline: 10.3847x; 1.1209x over previous
import functools

import jax
import jax.numpy as jnp
from jax import lax
from jax.experimental import pallas as pl
from jax.experimental.pallas import tpu as pltpu
from jax.experimental.pallas import tpu_sc as plsc

N = 10000
E = 320000
NC = 2
NS = 16
CHE = 128
EPT = 20480
NCHT = EPT // CHE
NPH = 4
E_PAD = NS * EPT
N_PAD = 10112
ZPT = N_PAD // NS
TPT = N // NS


def _make_agg(DH):
  D = 2 * DH
  mesh = plsc.VectorSubcoreMesh(core_axis_name="c", subcore_axis_name="s")

  @functools.partial(
      pl.kernel,
      out_type=jax.ShapeDtypeStruct((N_PAD, D), jnp.float32),
      mesh=mesh,
      compiler_params=pltpu.CompilerParams(use_tc_tiling_on_sc=False),
      scratch_types=[
          pltpu.VMEM((NCHT // NPH, CHE), jnp.int32),
          pltpu.VMEM((NCHT // NPH, CHE), jnp.int32),
          pltpu.VMEM((2, CHE, DH), jnp.float32),
          pltpu.VMEM_SHARED((N, DH), jnp.float32),
          pltpu.VMEM_SHARED((N_PAD, DH), jnp.float32),
          pltpu.SemaphoreType.DMA,
          pltpu.SemaphoreType.DMA,
          pltpu.SemaphoreType.DMA,
      ],
  )
  def agg(g_hbm, src_hbm, dst_hbm, out_hbm,
          srcv, dstv, rows, tab, acc, semt, sem0, sem1):
    c = lax.axis_index("c")
    s = lax.axis_index("s")
    nch_p = NCHT // NPH

    tcopy = pltpu.async_copy(
        g_hbm.at[pl.ds(s * TPT, TPT), pl.ds(c * DH, DH)],
        tab.at[pl.ds(s * TPT, TPT)], semt)
    z16 = jnp.zeros((16,), jnp.float32)

    def zz(j, carry):
      for l in range(DH // 16):
        rows[0, j, pl.ds(l * 16, 16)] = z16
      return carry

    lax.fori_loop(0, CHE, zz, 0)
    for r in range(ZPT // CHE):
      pltpu.sync_copy(rows.at[0], acc.at[pl.ds(s * ZPT + r * CHE, CHE)])
    rem = ZPT - (ZPT // CHE) * CHE
    pltpu.sync_copy(rows.at[0, pl.ds(0, rem)],
                    acc.at[pl.ds(s * ZPT + (ZPT // CHE) * CHE, rem)])
    tcopy.wait()
    plsc.subcore_barrier()

    for p in range(NPH):
      pltpu.sync_copy(src_hbm.at[s, pl.ds(p * nch_p, nch_p)], srcv)
      pltpu.sync_copy(dst_hbm.at[s, pl.ds(p * nch_p, nch_p)], dstv)
      pltpu.async_copy(tab.at[srcv.at[0]], rows.at[0], sem0)

      def step(i, carry):
        k0 = i * 2
        k1 = k0 + 1
        pltpu.async_copy(tab.at[srcv.at[k1]], rows.at[1], sem1)
        pltpu.make_async_copy(tab.at[srcv.at[k0]], rows.at[0], sem0).wait()
        pltpu.sync_copy(rows.at[0], acc.at[dstv.at[k0]], add=True)
        pltpu.async_copy(tab.at[srcv.at[k0 + 2]], rows.at[0], sem0)
        pltpu.make_async_copy(tab.at[srcv.at[k1]], rows.at[1], sem1).wait()
        pltpu.sync_copy(rows.at[1], acc.at[dstv.at[k1]], add=True)
        return carry

      lax.fori_loop(0, nch_p // 2 - 1, step, 0)

      kl = nch_p - 2
      pltpu.async_copy(tab.at[srcv.at[kl + 1]], rows.at[1], sem1)
      pltpu.make_async_copy(tab.at[srcv.at[kl]], rows.at[0], sem0).wait()
      pltpu.sync_copy(rows.at[0], acc.at[dstv.at[kl]], add=True)
      pltpu.make_async_copy(tab.at[srcv.at[kl + 1]], rows.at[1], sem1).wait()
      pltpu.sync_copy(rows.at[1], acc.at[dstv.at[kl + 1]], add=True)

    plsc.subcore_barrier()
    pltpu.sync_copy(acc.at[pl.ds(s * ZPT, ZPT)],
                    out_hbm.at[pl.ds(s * ZPT, ZPT), pl.ds(c * DH, DH)])

  return agg


_agg64 = _make_agg(64)
_agg32 = _make_agg(32)


def _mm_body(x_ref, w_ref, o_ref):
  o_ref[...] = jnp.dot(x_ref[...], w_ref[...],
                       preferred_element_type=jnp.float32)


def _combine_mm_body(g_ref, p_ref, b_ref, w_ref, o_ref):
  h = jnp.maximum(g_ref[...] + p_ref[...] + b_ref[...], 0.0)
  o_ref[...] = jnp.dot(h, w_ref[...], preferred_element_type=jnp.float32)


def _combine_body(g_ref, p_ref, b_ref, o_ref):
  o_ref[...] = g_ref[...] + p_ref[...] + b_ref[...]


_BM = 1000


def _mm(x, w):
  n, d = x.shape
  h = w.shape[1]
  return pl.pallas_call(
      _mm_body,
      grid=(n // _BM,),
      in_specs=[
          pl.BlockSpec((_BM, d), lambda i: (i, 0)),
          pl.BlockSpec((d, h), lambda i: (0, 0)),
      ],
      out_specs=pl.BlockSpec((_BM, h), lambda i: (i, 0)),
      out_shape=jax.ShapeDtypeStruct((n, h), jnp.float32),
  )(x, w)


def _combine_mm(g, p, b, w):
  n, d = g.shape
  h = w.shape[1]
  return pl.pallas_call(
      _combine_mm_body,
      grid=(n // _BM,),
      in_specs=[
          pl.BlockSpec((_BM, d), lambda i: (i, 0)),
          pl.BlockSpec((_BM, d), lambda i: (i, 0)),
          pl.BlockSpec((1, d), lambda i: (0, 0)),
          pl.BlockSpec((d, h), lambda i: (0, 0)),
      ],
      out_specs=pl.BlockSpec((_BM, h), lambda i: (i, 0)),
      out_shape=jax.ShapeDtypeStruct((n, h), jnp.float32),
  )(g, p, b.reshape(1, d), w)


def _combine(g, p, b):
  n, d = g.shape
  return pl.pallas_call(
      _combine_body,
      grid=(n // _BM,),
      in_specs=[
          pl.BlockSpec((_BM, d), lambda i: (i, 0)),
          pl.BlockSpec((_BM, d), lambda i: (i, 0)),
          pl.BlockSpec((1, d), lambda i: (0, 0)),
      ],
      out_specs=pl.BlockSpec((_BM, d), lambda i: (i, 0)),
      out_shape=jax.ShapeDtypeStruct((n, d), jnp.float32),
  )(g, p, b.reshape(1, d))


def kernel(x, edge_index, W1, b1, W2, b2, W3, b3):
  src = edge_index[0]
  dst = edge_index[1]
  npad = E_PAD - E
  srcp = jnp.concatenate([src, jnp.zeros((npad,), jnp.int32)]).reshape(
      NS, NCHT, CHE)
  dstp = jnp.concatenate([dst, jnp.full((npad,), N, jnp.int32)]).reshape(
      NS, NCHT, CHE)

  g1 = _mm(x, W1)
  p1 = _agg64(g1, srcp, dstp)
  g2 = _combine_mm(g1, p1, b1, W2)
  p2 = _agg64(g2, srcp, dstp)
  g3 = _combine_mm(g2, p2, b2, W3)
  p3 = _agg32(g3, srcp, dstp)
  return _combine(g3, p3, b3)


# speedup vs baseline: 12.5534x; 1.2088x over previous
import functools

import jax
import jax.numpy as jnp
from jax import lax
from jax.experimental import pallas as pl
from jax.experimental.pallas import tpu as pltpu
from jax.experimental.pallas import tpu_sc as plsc

N = 10000
E = 320000
NC = 2
NS = 16
CHE = 128
EPT = 20480
NCHT = EPT // CHE
NPH = 4
E_PAD = NS * EPT
N_PAD = 10112
ZPT = N_PAD // NS
TPT = N // NS


def _make_agg(DH):
  D = 2 * DH
  NB = 4
  mesh = plsc.VectorSubcoreMesh(core_axis_name="c", subcore_axis_name="s")

  @functools.partial(
      pl.kernel,
      out_type=jax.ShapeDtypeStruct((N_PAD, D), jnp.float32),
      mesh=mesh,
      compiler_params=pltpu.CompilerParams(use_tc_tiling_on_sc=False),
      scratch_types=[
          pltpu.VMEM((NCHT // NPH, CHE), jnp.int32),
          pltpu.VMEM((NCHT // NPH, CHE), jnp.int32),
          pltpu.VMEM((NB, CHE, DH), jnp.float32),
          pltpu.VMEM_SHARED((N_PAD, DH), jnp.float32),
          pltpu.VMEM_SHARED((N_PAD, DH), jnp.float32),
          pltpu.SemaphoreType.DMA,
      ] + [pltpu.SemaphoreType.DMA] * (2 * NB),
  )
  def agg(g_hbm, ep_hbm, out_hbm,
          srcv, dstv, rows, tab, acc, semt, *sems):
    sg = sems[:NB]
    ss = sems[NB:]
    c = lax.axis_index("c")
    s = lax.axis_index("s")
    nch_p = NCHT // NPH

    tcopy = pltpu.async_copy(
        g_hbm.at[pl.ds(s * TPT, TPT), pl.ds(c * DH, DH)],
        tab.at[pl.ds(s * TPT, TPT)], semt)
    z16 = jnp.zeros((16,), jnp.float32)

    def zz(j, carry):
      for l in range(DH // 16):
        rows[0, j, pl.ds(l * 16, 16)] = z16
      return carry

    lax.fori_loop(0, CHE, zz, 0)
    for r in range(ZPT // CHE):
      pltpu.sync_copy(rows.at[0], acc.at[pl.ds(s * ZPT + r * CHE, CHE)])
    rem = ZPT - (ZPT // CHE) * CHE
    pltpu.sync_copy(rows.at[0, pl.ds(0, rem)],
                    acc.at[pl.ds(s * ZPT + (ZPT // CHE) * CHE, rem)])
    tcopy.wait()
    plsc.subcore_barrier()

    def issue_gather(k, b):
      pltpu.async_copy(tab.at[srcv.at[k]], rows.at[b], sg[b])

    def wait_gather(k, b):
      pltpu.make_async_copy(tab.at[srcv.at[k]], rows.at[b], sg[b]).wait()

    def issue_scatter(k, b):
      pltpu.async_copy(rows.at[b], acc.at[dstv.at[k]], ss[b], add=True)

    def wait_scatter(b):
      pltpu.make_async_copy(rows.at[b], acc.at[dstv.at[0]], ss[b]).wait()

    def visit(v, b, swait, gissue):
      if swait:
        wait_scatter((b + 2) % NB)
      if gissue:
        issue_gather(v + 2, (b + 2) % NB)
      wait_gather(v, b)
      issue_scatter(v, b)

    for p in range(NPH):
      pltpu.sync_copy(ep_hbm.at[0, s, pl.ds(p * nch_p, nch_p)], srcv)
      pltpu.sync_copy(ep_hbm.at[1, s, pl.ds(p * nch_p, nch_p)], dstv)
      issue_gather(0, 0)
      issue_gather(1, 1)
      visit(0, 0, False, True)
      visit(1, 1, False, True)
      visit(2, 2, True, True)
      visit(3, 3, True, True)

      def step(i, carry):
        v0 = 4 + i * 4
        for j in range(NB):
          visit(v0 + j, j, True, True)
        return carry

      lax.fori_loop(0, (nch_p - 8) // 4, step, 0)
      visit(nch_p - 4, 0, True, True)
      visit(nch_p - 3, 1, True, True)
      visit(nch_p - 2, 2, True, False)
      visit(nch_p - 1, 3, True, False)
      wait_scatter(2)
      wait_scatter(3)

    plsc.subcore_barrier()
    pltpu.sync_copy(acc.at[pl.ds(s * ZPT, ZPT)],
                    out_hbm.at[pl.ds(s * ZPT, ZPT), pl.ds(c * DH, DH)])

  return agg


_agg64 = _make_agg(64)
_agg32 = _make_agg(32)


def _mm_body(x_ref, w_ref, o_ref):
  o_ref[...] = jnp.dot(x_ref[...], w_ref[...],
                       preferred_element_type=jnp.float32)


def _combine_mm_body(g_ref, p_ref, b_ref, w_ref, o_ref):
  h = jnp.maximum(g_ref[...] + p_ref[...] + b_ref[...], 0.0)
  o_ref[...] = jnp.dot(h, w_ref[...], preferred_element_type=jnp.float32)


def _combine_body(g_ref, p_ref, b_ref, o_ref):
  o_ref[...] = g_ref[...] + p_ref[...] + b_ref[...]


_BM = 1000


def _mm(x, w):
  n, d = x.shape
  h = w.shape[1]
  return pl.pallas_call(
      _mm_body,
      grid=(n // _BM,),
      in_specs=[
          pl.BlockSpec((_BM, d), lambda i: (i, 0)),
          pl.BlockSpec((d, h), lambda i: (0, 0)),
      ],
      out_specs=pl.BlockSpec((_BM, h), lambda i: (i, 0)),
      out_shape=jax.ShapeDtypeStruct((n, h), jnp.float32),
  )(x, w)


def _combine_mm(g, p, b, w):
  n, d = g.shape
  h = w.shape[1]
  return pl.pallas_call(
      _combine_mm_body,
      grid=(n // _BM,),
      in_specs=[
          pl.BlockSpec((_BM, d), lambda i: (i, 0)),
          pl.BlockSpec((_BM, d), lambda i: (i, 0)),
          pl.BlockSpec((1, d), lambda i: (0, 0)),
          pl.BlockSpec((d, h), lambda i: (0, 0)),
      ],
      out_specs=pl.BlockSpec((_BM, h), lambda i: (i, 0)),
      out_shape=jax.ShapeDtypeStruct((n, h), jnp.float32),
  )(g, p, b.reshape(1, d), w)


def _combine(g, p, b):
  n, d = g.shape
  return pl.pallas_call(
      _combine_body,
      grid=(n // _BM,),
      in_specs=[
          pl.BlockSpec((_BM, d), lambda i: (i, 0)),
          pl.BlockSpec((_BM, d), lambda i: (i, 0)),
          pl.BlockSpec((1, d), lambda i: (0, 0)),
      ],
      out_specs=pl.BlockSpec((_BM, d), lambda i: (i, 0)),
      out_shape=jax.ShapeDtypeStruct((n, d), jnp.float32),
  )(g, p, b.reshape(1, d))


def kernel(x, edge_index, W1, b1, W2, b2, W3, b3):
  ep = jnp.pad(edge_index, ((0, 0), (0, E_PAD - E)),
               constant_values=N).reshape(2, NS, NCHT, CHE)

  g1 = _mm(x, W1)
  p1 = _agg64(g1, ep)
  g2 = _combine_mm(g1, p1, b1, W2)
  p2 = _agg64(g2, ep)
  g3 = _combine_mm(g2, p2, b2, W3)
  p3 = _agg32(g3, ep)
  return _combine(g3, p3, b3)
